# baseline (speedup 1.0000x reference)
import functools

import jax
import jax.numpy as jnp
from jax import lax
from jax.experimental import pallas as pl
from jax.experimental.pallas import tpu as pltpu
from jax.experimental.pallas import tpu_sc as plsc

N = 19
E = 342
EP = 384
D_IN = 1025
HID = 256
OUT = 512
ROW = 128
AP = 2560
DUMP = N * ROW


def _build_a_body(ei_h, pc_h, out_h, ds_v, idx_v, pc_v, zbuf_v, acc_sh, sem):
    del sem
    cid = lax.axis_index("c")
    sid = lax.axis_index("s")

    @pl.when((cid == 0) & (sid == 0))
    def _():
        pltpu.sync_copy(ei_h, ds_v)
        pltpu.sync_copy(pc_h, pc_v.at[pl.ds(0, E)])
        zero = jnp.zeros((16,), jnp.float32)
        for i in range(AP // 16):
            zbuf_v[pl.ds(i * 16, 16)] = zero
        pltpu.sync_copy(zbuf_v, acc_sh)
        dump = jnp.full((16,), DUMP, jnp.int32)
        idx_v[2, pl.ds(86, 16)] = dump
        idx_v[2, pl.ds(102, 16)] = dump
        idx_v[2, pl.ds(112, 16)] = dump
        for c in range(22):
            off = c * 16 if c < 21 else E - 16
            d = ds_v[1, pl.ds(off, 16)]
            s = ds_v[0, pl.ds(off, 16)]
            idx_v[off // 128, pl.ds(off % 128, 16)] = d * ROW + s
        for j in range(EP // 128):
            pltpu.sync_copy(pc_v.at[pl.ds(j * 128, 128)],
                            acc_sh.at[idx_v.at[j]], add=True)
        pltpu.sync_copy(acc_sh, out_h)


_build_a = functools.partial(
    pl.kernel,
    out_type=jax.ShapeDtypeStruct((AP,), jnp.float32),
    mesh=plsc.VectorSubcoreMesh(core_axis_name="c", subcore_axis_name="s",
                                num_cores=1, num_subcores=1),
    scratch_types=[
        pltpu.VMEM((2, E), jnp.int32),
        pltpu.VMEM((EP // 128, 128), jnp.int32),
        pltpu.VMEM((EP,), jnp.float32),
        pltpu.VMEM((AP,), jnp.float32),
        pltpu.VMEM_SHARED((AP,), jnp.float32),
        pltpu.SemaphoreType.DMA,
    ],
)(_build_a_body)


def _proj_kernel(x_ref, w1a_ref, out_ref):
    out_ref[...] = jnp.dot(x_ref[...], w1a_ref[...],
                           preferred_element_type=jnp.float32)


def _dense_kernel(a_ref, p_ref,
                  b1a_ref, w1b_ref, b1b_ref,
                  w2a_ref, b2a_ref, w2b_ref, b2b_ref,
                  wp_ref, bp_ref, out_ref):
    f32 = jnp.float32
    a = a_ref[...].reshape(AP // ROW, ROW)[:N, :N]
    eye = (jax.lax.broadcasted_iota(jnp.int32, (N, N), 0)
           == jax.lax.broadcasted_iota(jnp.int32, (N, N), 1)).astype(f32)
    apl = a + eye

    t1 = jax.nn.relu(jnp.dot(apl, p_ref[...], preferred_element_type=f32)
                     + b1a_ref[...])
    h1 = jax.nn.relu(jnp.dot(t1, w1b_ref[...], preferred_element_type=f32)
                     + b1b_ref[...])

    z2 = jnp.dot(apl, h1, preferred_element_type=f32)
    t2 = jax.nn.relu(jnp.dot(z2, w2a_ref[...], preferred_element_type=f32)
                     + b2a_ref[...])
    h2 = jnp.dot(t2, w2b_ref[...], preferred_element_type=f32) + b2b_ref[...]

    s = jnp.sum(h2 * wp_ref[...], axis=1, keepdims=True)
    total = jnp.sum(s, axis=0, keepdims=True) + bp_ref[...]
    out_ref[...] = jax.nn.sigmoid(total)


@jax.jit
def _run(x, edge_index, pred_connectivity, W1a, b1a, W1b, b1b,
         W2a, b2a, W2b, b2b, Wp, bp):
    a_flat = _build_a(edge_index, pred_connectivity)
    p = pl.pallas_call(
        _proj_kernel,
        out_shape=jax.ShapeDtypeStruct((N, HID), jnp.float32),
    )(x, W1a)

    out = pl.pallas_call(
        _dense_kernel,
        out_shape=jax.ShapeDtypeStruct((1, 1), jnp.float32),
    )(a_flat, p, b1a, W1b, b1b, W2a, b2a, W2b, b2b, Wp.reshape(N, OUT), bp)
    return out.reshape(1)


def kernel(x, edge_index, pred_connectivity, W1a, b1a, W1b, b1b,
           W2a, b2a, W2b, b2b, Wp, bp):
    return _run(x, edge_index, pred_connectivity, W1a, b1a, W1b, b1b,
                W2a, b2a, W2b, b2b, Wp, bp)


# speedup vs baseline: 1.0442x; 1.0442x over previous
import functools

import jax
import jax.numpy as jnp
from jax import lax
from jax.experimental import pallas as pl
from jax.experimental.pallas import tpu as pltpu
from jax.experimental.pallas import tpu_sc as plsc

N = 19
E = 342
EP = 384
D_IN = 1025
HID = 256
OUT = 512
ROW = 128
AP = 2560
DUMP = N * ROW


def _build_a_body(ei_h, pc_h, out_h, ds_v, idx_v, pc_v, zbuf_v, acc_sh,
                  sem_ei, sem_pc, sem_z, sem_sc):
    cid = lax.axis_index("c")
    sid = lax.axis_index("s")

    @pl.when((cid == 0) & (sid == 0))
    def _():
        cp_ei = pltpu.async_copy(ei_h, ds_v, sem_ei)
        cp_pc = pltpu.async_copy(pc_h, pc_v.at[pl.ds(0, E)], sem_pc)
        zero = jnp.zeros((16,), jnp.float32)
        for i in range(AP // 16):
            zbuf_v[pl.ds(i * 16, 16)] = zero
        cp_z = pltpu.async_copy(zbuf_v, acc_sh, sem_z)
        dump = jnp.full((16,), DUMP, jnp.int32)
        idx_v[2, pl.ds(86, 16)] = dump
        idx_v[2, pl.ds(102, 16)] = dump
        idx_v[2, pl.ds(112, 16)] = dump
        cp_ei.wait()
        for c in range(22):
            off = c * 16 if c < 21 else E - 16
            d = ds_v[1, pl.ds(off, 16)]
            s = ds_v[0, pl.ds(off, 16)]
            idx_v[off // 128, pl.ds(off % 128, 16)] = d * ROW + s
        cp_pc.wait()
        cp_z.wait()
        sc = [pltpu.async_copy(pc_v.at[pl.ds(j * 128, 128)],
                               acc_sh.at[idx_v.at[j]], sem_sc, add=True)
              for j in range(EP // 128)]
        for c in sc:
            c.wait()
        pltpu.sync_copy(acc_sh, out_h)


_build_a = functools.partial(
    pl.kernel,
    out_type=jax.ShapeDtypeStruct((AP,), jnp.float32),
    mesh=plsc.VectorSubcoreMesh(core_axis_name="c", subcore_axis_name="s",
                                num_cores=1, num_subcores=1),
    scratch_types=[
        pltpu.VMEM((2, E), jnp.int32),
        pltpu.VMEM((EP // 128, 128), jnp.int32),
        pltpu.VMEM((EP,), jnp.float32),
        pltpu.VMEM((AP,), jnp.float32),
        pltpu.VMEM_SHARED((AP,), jnp.float32),
        pltpu.SemaphoreType.DMA,
        pltpu.SemaphoreType.DMA,
        pltpu.SemaphoreType.DMA,
        pltpu.SemaphoreType.DMA,
    ],
)(_build_a_body)


def _proj_kernel(x_ref, w1a_ref, out_ref):
    out_ref[...] = jnp.dot(x_ref[...], w1a_ref[...],
                           preferred_element_type=jnp.float32)


def _dense_kernel(a_ref, p_ref,
                  b1a_ref, w1b_ref, b1b_ref,
                  w2a_ref, b2a_ref, w2b_ref, b2b_ref,
                  wp_ref, bp_ref, out_ref):
    f32 = jnp.float32
    a = a_ref[...].reshape(AP // ROW, ROW)[:N, :N]
    eye = (jax.lax.broadcasted_iota(jnp.int32, (N, N), 0)
           == jax.lax.broadcasted_iota(jnp.int32, (N, N), 1)).astype(f32)
    apl = a + eye

    t1 = jax.nn.relu(jnp.dot(apl, p_ref[...], preferred_element_type=f32)
                     + b1a_ref[...])
    h1 = jax.nn.relu(jnp.dot(t1, w1b_ref[...], preferred_element_type=f32)
                     + b1b_ref[...])

    z2 = jnp.dot(apl, h1, preferred_element_type=f32)
    t2 = jax.nn.relu(jnp.dot(z2, w2a_ref[...], preferred_element_type=f32)
                     + b2a_ref[...])
    h2 = jnp.dot(t2, w2b_ref[...], preferred_element_type=f32) + b2b_ref[...]

    s = jnp.sum(h2 * wp_ref[...], axis=1, keepdims=True)
    total = jnp.sum(s, axis=0, keepdims=True) + bp_ref[...]
    out_ref[...] = jax.nn.sigmoid(total)


@jax.jit
def _run(x, edge_index, pred_connectivity, W1a, b1a, W1b, b1b,
         W2a, b2a, W2b, b2b, Wp, bp):
    a_flat = _build_a(edge_index, pred_connectivity)
    p = pl.pallas_call(
        _proj_kernel,
        out_shape=jax.ShapeDtypeStruct((N, HID), jnp.float32),
    )(x, W1a)

    out = pl.pallas_call(
        _dense_kernel,
        out_shape=jax.ShapeDtypeStruct((1, 1), jnp.float32),
    )(a_flat, p, b1a, W1b, b1b, W2a, b2a, W2b, b2b, Wp.reshape(N, OUT), bp)
    return out.reshape(1)


def kernel(x, edge_index, pred_connectivity, W1a, b1a, W1b, b1b,
           W2a, b2a, W2b, b2b, Wp, bp):
    return _run(x, edge_index, pred_connectivity, W1a, b1a, W1b, b1b,
                W2a, b2a, W2b, b2b, Wp, bp)
